# baseline (speedup 1.0000x reference)
import jax
import jax.numpy as jnp
from jax import lax
from jax.experimental import pallas as pl
from jax.experimental.pallas import tpu as pltpu
from jax.experimental.pallas import tpu_sc as plsc

B = 16384
N_FIELDS = 26
FIELD_VOCAB = 100000
UI_VOCAB = 1000000
EMB = 16
N_SLOTS = N_FIELDS + 2
R = N_SLOTS * EMB
NT = (R // 8) * (B // 2048)

_INFO = plsc.get_sparse_core_info()
NC = _INFO.num_cores
NS = _INFO.num_subcores
NW = NC * NS
TPW = NT // NW

def _gather_body(uids, iids, spidx, ut, it, st, out, ridx, idx2, dst, sem):
  wid = lax.axis_index("s") * NC + lax.axis_index("c")

  def do_task(idx_ref, idx_off, tab_ref, a_global, a_local, cb, vocab):
    pltpu.sync_copy(idx_ref.at[pl.ds(idx_off, 2048)], ridx)

    def build(bb, c):
      for s in range(8):
        base = (a_local * 8 + s) * vocab
        for kk in range(8):
          idx2[pl.ds(bb * 1024 + s * 128 + kk * 16, 16)] = (
              ridx[pl.ds(bb * 128 + kk * 16, 16)] + base)
      return c

    lax.fori_loop(0, 16, build, 0)
    pltpu.async_copy(tab_ref.at[idx2], dst, sem).wait()
    pltpu.sync_copy(dst,
                    out.at[pl.ds((a_global * 128 + cb * 16) * 1024, 16384)])

  @pl.when(wid < 16)
  def _():
    a = wid // 8
    cb = wid % 8
    do_task(uids, cb * 2048, ut, a, a, cb, UI_VOCAB)

  @pl.when(wid >= 16)
  def _():
    t = wid - 16
    a = t // 8
    cb = t % 8
    do_task(iids, cb * 2048, it, a + 2, a, cb, UI_VOCAB)

  def stask(k, carry):
    t = wid * 13 + k
    al = t // 8
    cb = t % 8
    do_task(spidx, (al // 2) * B + cb * 2048, st, al + 4, al, cb, FIELD_VOCAB)
    return carry

  lax.fori_loop(0, 13, stask, 0)


_gather = pl.kernel(
    _gather_body,
    out_type=jax.ShapeDtypeStruct((R * B,), jnp.float32),
    mesh=plsc.VectorSubcoreMesh(core_axis_name="c", subcore_axis_name="s"),
    scratch_types=[
        pltpu.VMEM((2048,), jnp.int32),
        pltpu.VMEM((16384,), jnp.int32),
        pltpu.VMEM((16384,), jnp.float32),
        pltpu.SemaphoreType.DMA,
    ],
    compiler_params=pltpu.CompilerParams(use_tc_tiling_on_sc=False),
)

NB = 16
GRID = B // (NB * 128)


def _head_body(x_ref, w1t_ref, b1_ref, w2t_ref, b2_ref, w3t_ref, b3_ref,
               w4t_ref, b4_ref, o_ref):
  for bi in range(NB):
    xb = x_ref[:, bi].reshape(R, 128)
    s = jnp.sum(xb, axis=0, keepdims=True)
    q = jnp.sum(xb * xb, axis=0, keepdims=True)
    h = jnp.maximum(
        jnp.dot(w1t_ref[...], xb, preferred_element_type=jnp.float32)
        + b1_ref[...], 0.0)
    h = jnp.maximum(
        jnp.dot(w2t_ref[...], h, preferred_element_type=jnp.float32)
        + b2_ref[...], 0.0)
    h = jnp.maximum(
        jnp.dot(w3t_ref[...], h, preferred_element_type=jnp.float32)
        + b3_ref[...], 0.0)
    d = (jnp.dot(w4t_ref[...], h, preferred_element_type=jnp.float32)
         + b4_ref[...])
    logit = d + 0.5 * (s * s - q)
    pos = 1.0 / (1.0 + jnp.exp(-logit))
    neg = jnp.exp(logit) / (1.0 + jnp.exp(logit))
    o_ref[0, pl.ds(bi * 128, 128)] = jnp.where(logit >= 0.0, pos, neg)[0]


_full = lambda i: (0, 0)
_head = pl.pallas_call(
    _head_body,
    grid=(GRID,),
    in_specs=[
        pl.BlockSpec((R // 8, NB, 8, 128), lambda i: (0, i, 0, 0)),
        pl.BlockSpec((EMB, R), _full),
        pl.BlockSpec((EMB, 1), _full),
        pl.BlockSpec((EMB, EMB), _full),
        pl.BlockSpec((EMB, 1), _full),
        pl.BlockSpec((EMB, EMB), _full),
        pl.BlockSpec((EMB, 1), _full),
        pl.BlockSpec((1, EMB), _full),
        pl.BlockSpec((1, 1), _full),
    ],
    out_specs=pl.BlockSpec((1, NB * 128), lambda i: (0, i)),
    out_shape=jax.ShapeDtypeStruct((1, B), jnp.float32),
)


@jax.jit
def kernel(user_ids, item_ids, sparse_features, user_table, item_table,
           sparse_tables, W1, b1, W2, b2, W3, b3, W4, b4):
  spidx = sparse_features.T.reshape(-1)
  ut = user_table.T.reshape(-1)
  it = item_table.T.reshape(-1)
  st = sparse_tables.transpose(0, 2, 1).reshape(-1)
  comb = _gather(user_ids, item_ids, spidx, ut, it, st)
  x4d = comb.reshape(R // 8, B // 128, 8, 128)
  out = _head(x4d,
              W1.T, b1.reshape(EMB, 1),
              W2.T, b2.reshape(EMB, 1),
              W3.T, b3.reshape(EMB, 1),
              W4.T, b4.reshape(1, 1))
  return out.reshape(B)


# speedup vs baseline: 1.7941x; 1.7941x over previous
import functools

import jax
import jax.numpy as jnp
from jax import lax
from jax.experimental import pallas as pl
from jax.experimental.pallas import tpu as pltpu
from jax.experimental.pallas import tpu_sc as plsc

B = 16384
N_FIELDS = 26
FIELD_VOCAB = 100000
EMB = 16
N_SLOTS = N_FIELDS + 2

_INFO = plsc.get_sparse_core_info()
NC = _INFO.num_cores
NS = _INFO.num_subcores
NW = NC * NS

ROWS_UI = B // NW
ROWS_SP = N_FIELDS * B // NW
CH = 512
SP_CHUNKS = ROWS_SP // CH


def _gather_body(uid, iid, spidx, ut, it, st, out, idx_v, rows_v, sem):
  wid = lax.axis_index("s") * NC + lax.axis_index("c")

  def one(idx_hbm, idx_off, table, out_off):
    pltpu.sync_copy(idx_hbm.at[pl.ds(idx_off, CH)], idx_v)
    pltpu.async_copy(table.at[idx_v], rows_v, sem).wait()
    pltpu.sync_copy(rows_v, out.at[pl.ds(out_off, CH)])

  base = wid * ROWS_UI
  one(uid, base, ut, base)
  one(iid, base, it, B + base)

  def body(j, carry):
    off = wid * ROWS_SP + j * CH
    one(spidx, off, st, 2 * B + off)
    return carry

  lax.fori_loop(0, SP_CHUNKS, body, 0)


_gather = pl.kernel(
    _gather_body,
    out_type=jax.ShapeDtypeStruct((N_SLOTS * B, EMB), jnp.float32),
    mesh=plsc.VectorSubcoreMesh(core_axis_name="c", subcore_axis_name="s"),
    scratch_types=[
        pltpu.VMEM((CH,), jnp.int32),
        pltpu.VMEM((CH, EMB), jnp.float32),
        pltpu.SemaphoreType.DMA,
    ],
    compiler_params=pltpu.CompilerParams(use_tc_tiling_on_sc=False),
)

BBG = 256
GRID = B // (BBG * 8)


def _head_body(x_ref, w1bd_ref, b1t_ref, w2bd_ref, b2t_ref, w3bd_ref,
               b3t_ref, w4bd_ref, b4_ref, ones_ref, o_ref):
  sv = jnp.zeros((BBG, 128), jnp.float32)
  qv = jnp.zeros((BBG, 128), jnp.float32)
  h0 = jnp.zeros((BBG, 128), jnp.float32)
  for f in range(N_SLOTS):
    xf = x_ref[f]
    sv = sv + xf
    qv = qv + xf * xf
    h0 = h0 + jnp.dot(xf, w1bd_ref[f], preferred_element_type=jnp.float32)
  srow = jnp.dot(sv, ones_ref[...], preferred_element_type=jnp.float32)
  qrow = jnp.dot(qv, ones_ref[...], preferred_element_type=jnp.float32)
  fm = 0.5 * (srow * srow - qrow)

  h = jnp.maximum(h0 + b1t_ref[...], 0.0)
  h = jnp.maximum(
      jnp.dot(h, w2bd_ref[...], preferred_element_type=jnp.float32)
      + b2t_ref[...], 0.0)
  h = jnp.maximum(
      jnp.dot(h, w3bd_ref[...], preferred_element_type=jnp.float32)
      + b3t_ref[...], 0.0)
  d = (jnp.dot(h, w4bd_ref[...], preferred_element_type=jnp.float32)
       + b4_ref[...])
  logit = d + fm
  pos = 1.0 / (1.0 + jnp.exp(-logit))
  neg = jnp.exp(logit) / (1.0 + jnp.exp(logit))
  o_ref[...] = jnp.where(logit >= 0.0, pos, neg)


_full = lambda i: (0, 0)
_head = pl.pallas_call(
    _head_body,
    grid=(GRID,),
    in_specs=[
        pl.BlockSpec((N_SLOTS, BBG, 128), lambda i: (0, i, 0)),
        pl.BlockSpec((N_SLOTS, 128, 128), lambda i: (0, 0, 0)),
        pl.BlockSpec((1, 128), _full),
        pl.BlockSpec((128, 128), _full),
        pl.BlockSpec((1, 128), _full),
        pl.BlockSpec((128, 128), _full),
        pl.BlockSpec((1, 128), _full),
        pl.BlockSpec((128, 8), _full),
        pl.BlockSpec((1, 1), _full),
        pl.BlockSpec((128, 8), _full),
    ],
    out_specs=pl.BlockSpec((BBG, 8), lambda i: (i, 0)),
    out_shape=jax.ShapeDtypeStruct((B // 8, 8), jnp.float32),
)


@jax.jit
def kernel(user_ids, item_ids, sparse_features, user_table, item_table,
           sparse_tables, W1, b1, W2, b2, W3, b3, W4, b4):
  offs = (jnp.arange(N_FIELDS, dtype=jnp.int32) * FIELD_VOCAB)[:, None]
  spidx = (sparse_features.T + offs).reshape(-1)
  spflat = sparse_tables.reshape(N_FIELDS * FIELD_VOCAB, EMB)
  comb = _gather(user_ids, item_ids, spidx, user_table, item_table, spflat)

  eye8 = jnp.eye(8, dtype=jnp.float32)
  kron = lambda w: jnp.kron(eye8, w)
  w1bd = jax.vmap(kron)(W1.reshape(N_SLOTS, EMB, EMB))
  out = _head(comb.reshape(N_SLOTS, B // 8, 128),
              w1bd, jnp.tile(b1, 8).reshape(1, 128),
              kron(W2), jnp.tile(b2, 8).reshape(1, 128),
              kron(W3), jnp.tile(b3, 8).reshape(1, 128),
              kron(W4), b4.reshape(1, 1),
              kron(jnp.ones((EMB, 1), jnp.float32)))
  return out.reshape(B)


# speedup vs baseline: 1.7942x; 1.0000x over previous
import jax
import jax.numpy as jnp
from jax import lax
from jax.experimental import pallas as pl
from jax.experimental.pallas import tpu as pltpu
from jax.experimental.pallas import tpu_sc as plsc

B = 16384
N_FIELDS = 26
FIELD_VOCAB = 100000
EMB = 16
N_SLOTS = N_FIELDS + 2

_INFO = plsc.get_sparse_core_info()
NC = _INFO.num_cores
NS = _INFO.num_subcores
NW = NC * NS

ROWS_UI = B // NW
ROWS_SP = N_FIELDS * B // NW
CH = 512
SP_CHUNKS = ROWS_SP // CH


def _gather_body(uid, iid, spidx, ut, it, st, out, idx_v, rows_v, sem):
  wid = lax.axis_index("s") * NC + lax.axis_index("c")

  def one(idx_hbm, idx_off, table, out_off):
    pltpu.sync_copy(idx_hbm.at[pl.ds(idx_off, CH)], idx_v)
    pltpu.async_copy(table.at[idx_v], rows_v, sem).wait()
    pltpu.sync_copy(rows_v, out.at[pl.ds(out_off, CH)])

  base = wid * ROWS_UI
  one(uid, base, ut, base)
  one(iid, base, it, B + base)

  def body(j, carry):
    off = wid * ROWS_SP + j * CH
    one(spidx, off, st, 2 * B + off)
    return carry

  lax.fori_loop(0, SP_CHUNKS, body, 0)


_gather = pl.kernel(
    _gather_body,
    out_type=jax.ShapeDtypeStruct((N_SLOTS * B, EMB), jnp.float32),
    mesh=plsc.VectorSubcoreMesh(core_axis_name="c", subcore_axis_name="s"),
    scratch_types=[
        pltpu.VMEM((CH,), jnp.int32),
        pltpu.VMEM((CH, EMB), jnp.float32),
        pltpu.SemaphoreType.DMA,
    ],
    compiler_params=pltpu.CompilerParams(use_tc_tiling_on_sc=False),
)

BBG = 256
GRID = B // (BBG * 8)


def _head_body(x_ref, w1bd_ref, b1t_ref, w2bd_ref, b2t_ref, w3bd_ref,
               b3t_ref, w4bd_ref, b4_ref, ones_ref, o_ref):
  sv = jnp.zeros((BBG, 128), jnp.float32)
  qv = jnp.zeros((BBG, 128), jnp.float32)
  h0 = jnp.zeros((BBG, 128), jnp.float32)
  for f in range(N_SLOTS):
    xf = x_ref[f]
    sv = sv + xf
    qv = qv + xf * xf
    h0 = h0 + jnp.dot(xf, w1bd_ref[f], preferred_element_type=jnp.float32)
  srow = jnp.dot(sv, ones_ref[...], preferred_element_type=jnp.float32)
  qrow = jnp.dot(qv, ones_ref[...], preferred_element_type=jnp.float32)
  fm = 0.5 * (srow * srow - qrow)

  h = jnp.maximum(h0 + b1t_ref[...], 0.0)
  h = jnp.maximum(
      jnp.dot(h, w2bd_ref[...], preferred_element_type=jnp.float32)
      + b2t_ref[...], 0.0)
  h = jnp.maximum(
      jnp.dot(h, w3bd_ref[...], preferred_element_type=jnp.float32)
      + b3t_ref[...], 0.0)
  d = (jnp.dot(h, w4bd_ref[...], preferred_element_type=jnp.float32)
       + b4_ref[...])
  logit = d + fm
  pos = 1.0 / (1.0 + jnp.exp(-logit))
  neg = jnp.exp(logit) / (1.0 + jnp.exp(logit))
  o_ref[...] = jnp.where(logit >= 0.0, pos, neg)


_full = lambda i: (0, 0)
_head = pl.pallas_call(
    _head_body,
    grid=(GRID,),
    in_specs=[
        pl.BlockSpec((N_SLOTS, BBG, 128), lambda i: (0, i, 0)),
        pl.BlockSpec((N_SLOTS, 128, 128), lambda i: (0, 0, 0)),
        pl.BlockSpec((1, 128), _full),
        pl.BlockSpec((128, 128), _full),
        pl.BlockSpec((1, 128), _full),
        pl.BlockSpec((128, 128), _full),
        pl.BlockSpec((1, 128), _full),
        pl.BlockSpec((128, 8), _full),
        pl.BlockSpec((1, 1), _full),
        pl.BlockSpec((128, 8), _full),
    ],
    out_specs=pl.BlockSpec((BBG, 8), lambda i: (i, 0)),
    out_shape=jax.ShapeDtypeStruct((B // 8, 8), jnp.float32),
)


@jax.jit
def kernel(user_ids, item_ids, sparse_features, user_table, item_table,
           sparse_tables, W1, b1, W2, b2, W3, b3, W4, b4):
  offs = (jnp.arange(N_FIELDS, dtype=jnp.int32) * FIELD_VOCAB)[:, None]
  spidx = (sparse_features.T + offs).reshape(-1)
  spflat = sparse_tables.reshape(N_FIELDS * FIELD_VOCAB, EMB)
  comb = _gather(user_ids, item_ids, spidx, user_table, item_table, spflat)

  eye8 = jnp.eye(8, dtype=jnp.float32)
  kron = lambda w: jnp.kron(eye8, w)
  w1bd = jax.vmap(kron)(W1.reshape(N_SLOTS, EMB, EMB))
  out = _head(comb.reshape(N_SLOTS, B // 8, 128),
              w1bd, jnp.tile(b1, 8).reshape(1, 128),
              kron(W2), jnp.tile(b2, 8).reshape(1, 128),
              kron(W3), jnp.tile(b3, 8).reshape(1, 128),
              kron(W4), b4.reshape(1, 1),
              kron(jnp.ones((EMB, 1), jnp.float32)))
  return out.reshape(B)
